# Initial kernel scaffold; baseline (speedup 1.0000x reference)
import jax
import jax.numpy as jnp
from jax.experimental import pallas as pl


def kernel(x):
    raise NotImplementedError("write your pallas kernel here")


# speedup vs baseline: 3.8855x; 3.8855x over previous
import jax
import jax.numpy as jnp
from jax.experimental import pallas as pl

_B, _C, _N, _M = 8, 3, 16384, 1024


def _fps_body(x0, x1, x2, y0, y1, y2):
    X0 = x0[...]
    X1 = x1[...]
    X2 = x2[...]
    lane = jax.lax.broadcasted_iota(jnp.int32, (_B, _N), 1)
    out_lane = jax.lax.broadcasted_iota(jnp.int32, (_B, _M), 1)

    def step(k, carry):
        dist, a, b, c, Y0, Y1, Y2 = carry
        sel = out_lane == k
        Y0 = jnp.where(sel, a, Y0)
        Y1 = jnp.where(sel, b, Y1)
        Y2 = jnp.where(sel, c, Y2)
        s0 = X0 - a
        s1 = X1 - b
        s2 = X2 - c
        d = (s0 * s0 + s1 * s1) + s2 * s2
        dist = jnp.minimum(dist, d)
        mx = jnp.max(dist, axis=1, keepdims=True)
        cand = jnp.where(dist == mx, lane, _N)
        idx = jnp.min(cand, axis=1, keepdims=True)
        w = lane == idx
        a = jnp.sum(jnp.where(w, X0, 0.0), axis=1, keepdims=True)
        b = jnp.sum(jnp.where(w, X1, 0.0), axis=1, keepdims=True)
        c = jnp.sum(jnp.where(w, X2, 0.0), axis=1, keepdims=True)
        return dist, a, b, c, Y0, Y1, Y2

    dist0 = jnp.full((_B, _N), jnp.inf, dtype=jnp.float32)
    z = jnp.zeros((_B, _M), dtype=jnp.float32)
    init = (dist0, X0[:, 0:1], X1[:, 0:1], X2[:, 0:1], z, z, z)
    _, _, _, _, Y0, Y1, Y2 = jax.lax.fori_loop(0, _M, step, init)
    y0[...] = Y0
    y1[...] = Y1
    y2[...] = Y2


def kernel(x):
    x0 = x[:, 0, :]
    x1 = x[:, 1, :]
    x2 = x[:, 2, :]
    y0, y1, y2 = pl.pallas_call(
        _fps_body,
        out_shape=[jax.ShapeDtypeStruct((_B, _M), jnp.float32)] * 3,
    )(x0, x1, x2)
    return jnp.stack([y0, y1, y2], axis=1)


# speedup vs baseline: 5.6907x; 1.4646x over previous
import jax
import jax.numpy as jnp
from jax.experimental import pallas as pl
from jax.experimental.pallas import tpu as pltpu

_B, _C, _N, _M = 8, 3, 16384, 1024
_CH = 128
_NCH = _N // _CH
_NBANK = 2
_W = 128


def _fps_body(x0, x1, x2, y0, y1, y2, dist_ref):
    lane = jax.lax.broadcasted_iota(jnp.int32, (_B, _CH), 1)
    win_lane = jax.lax.broadcasted_iota(jnp.int32, (_B, _W), 1)
    dist_ref[...] = jnp.full((_B, _N), jnp.inf, dtype=jnp.float32)

    def step(k, carry):
        a, b, c, W0, W1, W2 = carry
        selk = win_lane == jax.lax.rem(k, _W)
        W0 = jnp.where(selk, a, W0)
        W1 = jnp.where(selk, b, W1)
        W2 = jnp.where(selk, c, W2)
        def half_scan(c_lo, c_hi):
            banks = []
            per = (c_hi - c_lo) // _NBANK
            for g in range(_NBANK):
                runmax = jnp.full((_B, _CH), -jnp.inf, dtype=jnp.float32)
                runidx = jnp.zeros((_B, _CH), dtype=jnp.int32)
                runa = jnp.zeros((_B, _CH), dtype=jnp.float32)
                runb = jnp.zeros((_B, _CH), dtype=jnp.float32)
                runc = jnp.zeros((_B, _CH), dtype=jnp.float32)
                for j in range(c_lo + g * per, c_lo + (g + 1) * per):
                    sl = pl.ds(j * _CH, _CH)
                    X0j = x0[:, sl]
                    X1j = x1[:, sl]
                    X2j = x2[:, sl]
                    s0 = X0j - a
                    s1 = X1j - b
                    s2 = X2j - c
                    d = (s0 * s0 + s1 * s1) + s2 * s2
                    dj = jnp.minimum(dist_ref[:, sl], d)
                    dist_ref[:, sl] = dj
                    upd = dj > runmax
                    runmax = jnp.where(upd, dj, runmax)
                    runidx = jnp.where(upd, j, runidx)
                    runa = jnp.where(upd, X0j, runa)
                    runb = jnp.where(upd, X1j, runb)
                    runc = jnp.where(upd, X2j, runc)
                banks.append((runmax, runidx, runa, runb, runc))
            runmax, runidx, runa, runb, runc = banks[0]
            for g in range(1, _NBANK):
                m2, i2, a2, b2, c2 = banks[g]
                upd = m2 > runmax
                runmax = jnp.where(upd, m2, runmax)
                runidx = jnp.where(upd, i2, runidx)
                runa = jnp.where(upd, a2, runa)
                runb = jnp.where(upd, b2, runb)
                runc = jnp.where(upd, c2, runc)
            mx = jnp.max(runmax, axis=1, keepdims=True)
            comp = runidx * _CH + lane
            cand = jnp.where(runmax == mx, comp, _N)
            idx = jnp.min(cand, axis=1, keepdims=True)
            w = cand == idx
            ra = jnp.sum(jnp.where(w, runa, 0.0), axis=1, keepdims=True)
            rb = jnp.sum(jnp.where(w, runb, 0.0), axis=1, keepdims=True)
            rc = jnp.sum(jnp.where(w, runc, 0.0), axis=1, keepdims=True)
            return mx, ra, rb, rc

        mxA, aA, bA, cA = half_scan(0, _NCH // 2)
        mxB, aB, bB, cB = half_scan(_NCH // 2, _NCH)
        updB = mxB > mxA
        a = jnp.where(updB, aB, aA)
        b = jnp.where(updB, bB, bA)
        c = jnp.where(updB, cB, cA)
        return a, b, c, W0, W1, W2

    z = jnp.zeros((_B, _W), dtype=jnp.float32)
    carry = (x0[:, 0:1], x1[:, 0:1], x2[:, 0:1], z, z, z)
    for o in range(_M // _W):
        carry = jax.lax.fori_loop(o * _W, (o + 1) * _W, step, carry)
        a, b, c, W0, W1, W2 = carry
        sl = slice(o * _W, (o + 1) * _W)
        y0[:, sl] = W0
        y1[:, sl] = W1
        y2[:, sl] = W2
        carry = (a, b, c, z, z, z)


def kernel(x):
    x0 = x[:, 0, :]
    x1 = x[:, 1, :]
    x2 = x[:, 2, :]
    y0, y1, y2 = pl.pallas_call(
        _fps_body,
        out_shape=[jax.ShapeDtypeStruct((_B, _M), jnp.float32)] * 3,
        scratch_shapes=[pltpu.VMEM((_B, _N), jnp.float32)],
    )(x0, x1, x2)
    return jnp.stack([y0, y1, y2], axis=1)
